# baseline (speedup 1.0000x reference)
import functools


import jax
import jax.numpy as jnp
from jax import lax
from jax.experimental import pallas as pl
from jax.experimental.pallas import tpu as pltpu
from jax.experimental.pallas import tpu_sc as plsc

H = 64
L = 16
NC = 2
NS = 16
NW = NC * NS
SB = 80


def _node_prep_body(nf, w, b, wtop, ns_o, p_o):
    ns = jnp.maximum(jnp.dot(nf[...], w[...], preferred_element_type=jnp.float32)
                     + b[...], 0.0)
    ns_o[...] = ns
    p = jnp.dot(ns, wtop[...], preferred_element_type=jnp.float32)
    p_o[...] = jnp.concatenate([p, jnp.zeros_like(p)], axis=1)


def _edge_prep_body(ef_lo, ef_hi, we, be, wm, bm, ec_o):
    es_lo = jnp.maximum(
        jnp.dot(ef_lo[...], we[...], preferred_element_type=jnp.float32)
        + be[...], 0.0)
    es_hi = jnp.maximum(
        jnp.dot(ef_hi[...], we[...], preferred_element_type=jnp.float32)
        + be[...], 0.0)
    lo = jnp.dot(es_lo, wm[...], preferred_element_type=jnp.float32) + bm[...]
    hi = jnp.dot(es_hi, wm[...], preferred_element_type=jnp.float32) + bm[...]
    ec_o[...] = jnp.concatenate([lo, hi], axis=1)


def _update_body(ns, parts, wu_t, wu_b, bu, wtop, ns_o, p_o):
    agg = parts[0, :, :H] + parts[1, :, :H]
    x = (jnp.dot(ns[...], wu_t[...], preferred_element_type=jnp.float32)
         + jnp.dot(agg, wu_b[...], preferred_element_type=jnp.float32) + bu[...])
    ns2 = jnp.maximum(x, 0.0)
    ns_o[...] = ns2
    p = jnp.dot(ns2, wtop[...], preferred_element_type=jnp.float32)
    p_o[...] = jnp.concatenate([p, jnp.zeros_like(p)], axis=1)


def _final_body(ns, parts, wu_t, wu_b, bu, ns_o):
    agg = parts[0, :, :H] + parts[1, :, :H]
    x = (jnp.dot(ns[...], wu_t[...], preferred_element_type=jnp.float32)
         + jnp.dot(agg, wu_b[...], preferred_element_type=jnp.float32) + bu[...])
    ns_o[...] = jnp.maximum(x, 0.0)


def _make_sc_round(n_nodes, n_edges):
    edges_per_tile = n_edges // NW
    rows_per_tile = n_nodes // NS
    mesh = plsc.VectorSubcoreMesh(core_axis_name="c", subcore_axis_name="s",
                                  num_cores=NC, num_subcores=NS)

    n_iter = edges_per_tile // SB
    PR = SB // 2

    @functools.partial(
        pl.kernel, mesh=mesh,
        compiler_params=pltpu.CompilerParams(needs_layout_passes=False),
        out_type=jax.ShapeDtypeStruct((NW, n_nodes // NS, 2 * H), jnp.float32),
        scratch_types=[
            pltpu.VMEM_SHARED((n_nodes, 2 * H), jnp.float32),
            [pltpu.VMEM((PR, 2 * H), jnp.float32) for _ in range(2)],
            [pltpu.VMEM((SB, 2 * H), jnp.float32) for _ in range(2)],
            [pltpu.VMEM((SB,), jnp.int32) for _ in range(2)],
            [pltpu.VMEM((SB,), jnp.int32) for _ in range(4)],
            [pltpu.SemaphoreType.DMA for _ in range(2)],
            [pltpu.SemaphoreType.DMA for _ in range(2)],
            [pltpu.SemaphoreType.DMA for _ in range(2)],
            [pltpu.SemaphoreType.DMA for _ in range(2)],
        ],
    )
    def sc_round(p_hbm, ec_hbm, src_hbm, dst_hbm, out_hbm,
                 agg_sh, ec_v, rows_v, sidx, didx,
                 sem_ec, sem_g, sem_i, sem_s):
        cid = lax.axis_index("c")
        sid = lax.axis_index("s")
        wid = sid * NC + cid

        zero = jnp.zeros((L,), jnp.float32)

        def _zstep(i, carry):
            for j in range(2 * H // L):
                rows_v[0][i, pl.ds(j * L, L)] = zero
            return carry

        lax.fori_loop(0, SB, _zstep, 0)
        node_base = sid * rows_per_tile
        done = 0
        while done < rows_per_tile:
            n = min(SB, rows_per_tile - done)
            pltpu.sync_copy(rows_v[0].at[pl.ds(0, n)],
                            agg_sh.at[pl.ds(node_base + done, n)])
            done += n
        plsc.subcore_barrier()

        edge0 = wid * edges_per_tile
        pair0 = wid * (edges_per_tile // 2)

        def issue_stage(k, b, d):
            pltpu.async_copy(src_hbm.at[pl.ds(edge0 + k * SB, SB)],
                             sidx[b], sem_i[b])
            pltpu.async_copy(dst_hbm.at[pl.ds(edge0 + k * SB, SB)],
                             didx[d], sem_i[b])
            pltpu.async_copy(ec_hbm.at[pl.ds(pair0 + k * PR, PR)],
                             ec_v[b], sem_ec[b])

        def wait_stage_idx(b):
            pltpu.make_async_copy(src_hbm.at[pl.ds(0, SB)], sidx[b],
                                  sem_i[b]).wait()
            pltpu.make_async_copy(dst_hbm.at[pl.ds(0, SB)], didx[0],
                                  sem_i[b]).wait()

        def wait_ec(b):
            pltpu.make_async_copy(ec_hbm.at[pl.ds(0, PR)], ec_v[b],
                                  sem_ec[b]).wait()

        def issue_gather(b):
            pltpu.async_copy(p_hbm.at[sidx[b]], rows_v[b], sem_g[b])

        def wait_gather(b):
            pltpu.make_async_copy(p_hbm.at[sidx[b]], rows_v[b],
                                  sem_g[b]).wait()

        def issue_scatter(b, d):
            pltpu.async_copy(rows_v[b], agg_sh.at[didx[d]], sem_s[b], add=True)

        def wait_scatter(b):
            pltpu.make_async_copy(rows_v[b], agg_sh.at[didx[0]],
                                  sem_s[b]).wait()

        def compute(b):
            ecb = ec_v[b]
            rb = rows_v[b]

            @plsc.parallel_loop(0, PR, 1, unroll=4)
            def _row(i2):
                for jj in range(2 * H // L):
                    se = pl.ds(jj * L, L)
                    sr = pl.ds((jj % 4) * L, L)
                    r = i2 + PR * (jj // 4)
                    rb[r, sr] = jnp.maximum(ecb[i2, se] + rb[r, sr], 0.0)

        def iteration(k, b, d, d2, first=False, guard=True):
            b1 = 1 - b
            wait_ec(b)
            wait_gather(b)
            compute(b)
            issue_scatter(b, d)

            def _prep_next():
                wait_stage_idx(b1)
                if not first:
                    wait_scatter(b1)
                issue_gather(b1)

            def _stage_next2():
                issue_stage(k + 2, b, d2)

            if guard:
                pl.when(k + 1 < n_iter)(_prep_next)
                pl.when(k + 2 < n_iter)(_stage_next2)
            else:
                _prep_next()
                _stage_next2()

        issue_stage(0, 0, 0)
        wait_stage_idx(0)
        issue_gather(0)
        issue_stage(1, 1, 1)
        iteration(0, 0, 0, 2, first=True, guard=False)

        def _quad(i, carry):
            k = 4 * i + 1
            iteration(k, 1, 1, 3)
            iteration(k + 1, 0, 2, 0)
            iteration(k + 2, 1, 3, 1)
            iteration(k + 3, 0, 0, 2)
            return carry

        lax.fori_loop(0, (n_iter - 1) // 4, _quad, 0)
        wait_scatter(1)
        wait_scatter(0)
        plsc.subcore_barrier()

        pltpu.sync_copy(agg_sh.at[pl.ds(node_base, rows_per_tile)],
                        out_hbm.at[cid * NS + sid])

    return sc_round


def kernel(node_features, edge_index, edge_features,
           W_node_in, b_node_in, W_edge_in, b_edge_in,
           W_m0, b_m0, W_m1, b_m1, W_u0, b_u0, W_u1, b_u1):
    n_nodes, _ = node_features.shape
    n_edges = edge_index.shape[1]
    f32 = jnp.float32

    half = n_edges // 2
    n_it = (n_edges // NW) // SB
    PR = SB // 2

    def _shuffle(ix):
        lo = ix[:half].reshape(NW, n_it, PR)
        hi = ix[half:].reshape(NW, n_it, PR)
        return jnp.concatenate([lo, hi], axis=2).reshape(-1)

    src = _shuffle(edge_index[0])
    dst = _shuffle(edge_index[1])
    b_node = b_node_in.reshape(1, H)
    b_edge = b_edge_in.reshape(1, H)
    bm0 = b_m0.reshape(1, H)
    bm1 = b_m1.reshape(1, H)
    bu0 = b_u0.reshape(1, H)
    bu1 = b_u1.reshape(1, H)
    d_edge = edge_features.shape[1]

    wm0_top = W_m0[:H]
    wm1_top = W_m1[:H]

    ns0, p0 = pl.pallas_call(
        _node_prep_body,
        out_shape=(jax.ShapeDtypeStruct((n_nodes, H), f32),
                   jax.ShapeDtypeStruct((n_nodes, 2 * H), f32)),
    )(node_features, W_node_in, b_node, wm0_top)

    eblk = 8000
    n_pairs = n_edges // 2
    grid = n_pairs // eblk
    nblk = grid

    def _edge_prep(wm_bot, bm_r):
        return pl.pallas_call(
            _edge_prep_body,
            grid=(grid,),
            in_specs=[
                pl.BlockSpec((eblk, d_edge), lambda i: (i, 0)),
                pl.BlockSpec((eblk, d_edge), lambda i: (i + nblk, 0)),
                pl.BlockSpec((d_edge, H), lambda i: (0, 0)),
                pl.BlockSpec((1, H), lambda i: (0, 0)),
                pl.BlockSpec((H, H), lambda i: (0, 0)),
                pl.BlockSpec((1, H), lambda i: (0, 0)),
            ],
            out_specs=pl.BlockSpec((eblk, 2 * H), lambda i: (i, 0)),
            out_shape=jax.ShapeDtypeStruct((n_pairs, 2 * H), f32),
        )(edge_features, edge_features, W_edge_in, b_edge, wm_bot, bm_r)

    ec0 = _edge_prep(W_m0[H:], bm0)
    ec1 = _edge_prep(W_m1[H:], bm1)

    sc_round = _make_sc_round(n_nodes, n_edges)

    parts0 = sc_round(p0, ec0, src, dst).reshape(NC, n_nodes, 2 * H)

    ns1, p1 = pl.pallas_call(
        _update_body,
        out_shape=(jax.ShapeDtypeStruct((n_nodes, H), f32),
                   jax.ShapeDtypeStruct((n_nodes, 2 * H), f32)),
    )(ns0, parts0, W_u0[:H], W_u0[H:], bu0, wm1_top)

    parts1 = sc_round(p1, ec1, src, dst).reshape(NC, n_nodes, 2 * H)

    ns2 = pl.pallas_call(
        _final_body,
        out_shape=jax.ShapeDtypeStruct((n_nodes, H), f32),
    )(ns1, parts1, W_u1[:H], W_u1[H:], bu1)

    return ns2


# speedup vs baseline: 1.0470x; 1.0470x over previous
import functools


import jax
import jax.numpy as jnp
from jax import lax
from jax.experimental import pallas as pl
from jax.experimental.pallas import tpu as pltpu
from jax.experimental.pallas import tpu_sc as plsc

H = 64
L = 16
NC = 2
NS = 16
NW = NC * NS
SB = 80


def _node_prep_body(nf, w, b, wtop, ns_o, p_o):
    ns = jnp.maximum(jnp.dot(nf[...], w[...], preferred_element_type=jnp.float32)
                     + b[...], 0.0)
    ns_o[...] = ns
    p = jnp.dot(ns, wtop[...], preferred_element_type=jnp.float32)
    p_o[...] = jnp.concatenate([p, jnp.zeros_like(p)], axis=1)


def _edge_prep_body(ef_lo, ef_hi, we, be, wm, bm, ec_o):
    es_lo = jnp.maximum(
        jnp.dot(ef_lo[...], we[...], preferred_element_type=jnp.float32)
        + be[...], 0.0)
    es_hi = jnp.maximum(
        jnp.dot(ef_hi[...], we[...], preferred_element_type=jnp.float32)
        + be[...], 0.0)
    lo = jnp.dot(es_lo, wm[...], preferred_element_type=jnp.float32) + bm[...]
    hi = jnp.dot(es_hi, wm[...], preferred_element_type=jnp.float32) + bm[...]
    ec_o[...] = jnp.concatenate([lo, hi], axis=1)


def _update_body(ns, parts, wu_t, wu_b, bu, wtop, ns_o, p_o):
    agg = parts[0, :, :H] + parts[1, :, :H]
    x = (jnp.dot(ns[...], wu_t[...], preferred_element_type=jnp.float32)
         + jnp.dot(agg, wu_b[...], preferred_element_type=jnp.float32) + bu[...])
    ns2 = jnp.maximum(x, 0.0)
    ns_o[...] = ns2
    p = jnp.dot(ns2, wtop[...], preferred_element_type=jnp.float32)
    p_o[...] = jnp.concatenate([p, jnp.zeros_like(p)], axis=1)


def _final_body(ns, parts, wu_t, wu_b, bu, ns_o):
    agg = parts[0, :, :H] + parts[1, :, :H]
    x = (jnp.dot(ns[...], wu_t[...], preferred_element_type=jnp.float32)
         + jnp.dot(agg, wu_b[...], preferred_element_type=jnp.float32) + bu[...])
    ns_o[...] = jnp.maximum(x, 0.0)


def _make_sc_round(n_nodes, n_edges):
    edges_per_tile = n_edges // NW
    rows_per_tile = n_nodes // NS
    mesh = plsc.VectorSubcoreMesh(core_axis_name="c", subcore_axis_name="s",
                                  num_cores=NC, num_subcores=NS)

    n_iter = edges_per_tile // SB
    PR = SB // 2

    @functools.partial(
        pl.kernel, mesh=mesh,
        compiler_params=pltpu.CompilerParams(needs_layout_passes=False),
        out_type=jax.ShapeDtypeStruct((NW, n_nodes // NS, 2 * H), jnp.float32),
        scratch_types=[
            pltpu.VMEM_SHARED((n_nodes, 2 * H), jnp.float32),
            [pltpu.VMEM((PR, 2 * H), jnp.float32) for _ in range(2)],
            [pltpu.VMEM((SB, 2 * H), jnp.float32) for _ in range(2)],
            [pltpu.VMEM((SB,), jnp.int32) for _ in range(2)],
            [pltpu.VMEM((SB,), jnp.int32) for _ in range(4)],
            [pltpu.SemaphoreType.DMA for _ in range(2)],
            [pltpu.SemaphoreType.DMA for _ in range(2)],
            [pltpu.SemaphoreType.DMA for _ in range(2)],
            [pltpu.SemaphoreType.DMA for _ in range(2)],
        ],
    )
    def sc_round(p_hbm, ec_hbm, src_hbm, dst_hbm, out_hbm,
                 agg_sh, ec_v, rows_v, sidx, didx,
                 sem_ec, sem_g, sem_i, sem_s):
        cid = lax.axis_index("c")
        sid = lax.axis_index("s")
        wid = sid * NC + cid

        zero = jnp.zeros((L,), jnp.float32)

        def _zstep(i, carry):
            for j in range(2 * H // L):
                rows_v[0][i, pl.ds(j * L, L)] = zero
            return carry

        lax.fori_loop(0, SB, _zstep, 0)
        node_base = sid * rows_per_tile
        done = 0
        while done < rows_per_tile:
            n = min(SB, rows_per_tile - done)
            pltpu.sync_copy(rows_v[0].at[pl.ds(0, n)],
                            agg_sh.at[pl.ds(node_base + done, n)])
            done += n
        plsc.subcore_barrier()

        pair0 = wid * (edges_per_tile // 2)
        half = n_edges // 2

        def issue_stage(k, b, d):
            lo = pair0 + k * PR
            pltpu.async_copy(src_hbm.at[pl.ds(lo, PR)],
                             sidx[b].at[pl.ds(0, PR)], sem_i[b])
            pltpu.async_copy(src_hbm.at[pl.ds(half + lo, PR)],
                             sidx[b].at[pl.ds(PR, PR)], sem_i[b])
            pltpu.async_copy(dst_hbm.at[pl.ds(lo, PR)],
                             didx[d].at[pl.ds(0, PR)], sem_i[b])
            pltpu.async_copy(dst_hbm.at[pl.ds(half + lo, PR)],
                             didx[d].at[pl.ds(PR, PR)], sem_i[b])
            pltpu.async_copy(ec_hbm.at[pl.ds(lo, PR)], ec_v[b], sem_ec[b])

        def wait_stage_idx(b):
            pltpu.make_async_copy(src_hbm.at[pl.ds(0, SB)], sidx[b],
                                  sem_i[b]).wait()
            pltpu.make_async_copy(dst_hbm.at[pl.ds(0, SB)], didx[0],
                                  sem_i[b]).wait()

        def wait_ec(b):
            pltpu.make_async_copy(ec_hbm.at[pl.ds(0, PR)], ec_v[b],
                                  sem_ec[b]).wait()

        def issue_gather(b):
            pltpu.async_copy(p_hbm.at[sidx[b]], rows_v[b], sem_g[b])

        def wait_gather(b):
            pltpu.make_async_copy(p_hbm.at[sidx[b]], rows_v[b],
                                  sem_g[b]).wait()

        def issue_scatter(b, d):
            pltpu.async_copy(rows_v[b], agg_sh.at[didx[d]], sem_s[b], add=True)

        def wait_scatter(b):
            pltpu.make_async_copy(rows_v[b], agg_sh.at[didx[0]],
                                  sem_s[b]).wait()

        def compute(b):
            ecb = ec_v[b]
            rb = rows_v[b]

            @plsc.parallel_loop(0, PR, 1, unroll=8)
            def _row(i2):
                for jj in range(2 * H // L):
                    se = pl.ds(jj * L, L)
                    sr = pl.ds((jj % 4) * L, L)
                    r = i2 + PR * (jj // 4)
                    rb[r, sr] = jnp.maximum(ecb[i2, se] + rb[r, sr], 0.0)

        def iteration(k, b, d, d2, first=False, guard=True):
            b1 = 1 - b
            wait_ec(b)
            wait_gather(b)
            compute(b)
            issue_scatter(b, d)

            def _prep_next():
                wait_stage_idx(b1)
                if not first:
                    wait_scatter(b1)
                issue_gather(b1)

            def _stage_next2():
                issue_stage(k + 2, b, d2)

            if guard:
                pl.when(k + 1 < n_iter)(_prep_next)
                pl.when(k + 2 < n_iter)(_stage_next2)
            else:
                _prep_next()
                _stage_next2()

        issue_stage(0, 0, 0)
        wait_stage_idx(0)
        issue_gather(0)
        issue_stage(1, 1, 1)
        iteration(0, 0, 0, 2, first=True, guard=False)

        def _quad(i, carry):
            k = 4 * i + 1
            iteration(k, 1, 1, 3)
            iteration(k + 1, 0, 2, 0)
            iteration(k + 2, 1, 3, 1)
            iteration(k + 3, 0, 0, 2)
            return carry

        lax.fori_loop(0, (n_iter - 1) // 4, _quad, 0)
        wait_scatter(1)
        wait_scatter(0)
        plsc.subcore_barrier()

        pltpu.sync_copy(agg_sh.at[pl.ds(node_base, rows_per_tile)],
                        out_hbm.at[cid * NS + sid])

    return sc_round


def kernel(node_features, edge_index, edge_features,
           W_node_in, b_node_in, W_edge_in, b_edge_in,
           W_m0, b_m0, W_m1, b_m1, W_u0, b_u0, W_u1, b_u1):
    n_nodes, _ = node_features.shape
    n_edges = edge_index.shape[1]
    f32 = jnp.float32

    src = edge_index[0]
    dst = edge_index[1]
    b_node = b_node_in.reshape(1, H)
    b_edge = b_edge_in.reshape(1, H)
    bm0 = b_m0.reshape(1, H)
    bm1 = b_m1.reshape(1, H)
    bu0 = b_u0.reshape(1, H)
    bu1 = b_u1.reshape(1, H)
    d_edge = edge_features.shape[1]

    wm0_top = W_m0[:H]
    wm1_top = W_m1[:H]

    ns0, p0 = pl.pallas_call(
        _node_prep_body,
        out_shape=(jax.ShapeDtypeStruct((n_nodes, H), f32),
                   jax.ShapeDtypeStruct((n_nodes, 2 * H), f32)),
    )(node_features, W_node_in, b_node, wm0_top)

    eblk = 8000
    n_pairs = n_edges // 2
    grid = n_pairs // eblk
    nblk = grid

    def _edge_prep(wm_bot, bm_r):
        return pl.pallas_call(
            _edge_prep_body,
            grid=(grid,),
            in_specs=[
                pl.BlockSpec((eblk, d_edge), lambda i: (i, 0)),
                pl.BlockSpec((eblk, d_edge), lambda i: (i + nblk, 0)),
                pl.BlockSpec((d_edge, H), lambda i: (0, 0)),
                pl.BlockSpec((1, H), lambda i: (0, 0)),
                pl.BlockSpec((H, H), lambda i: (0, 0)),
                pl.BlockSpec((1, H), lambda i: (0, 0)),
            ],
            out_specs=pl.BlockSpec((eblk, 2 * H), lambda i: (i, 0)),
            out_shape=jax.ShapeDtypeStruct((n_pairs, 2 * H), f32),
        )(edge_features, edge_features, W_edge_in, b_edge, wm_bot, bm_r)

    ec0 = _edge_prep(W_m0[H:], bm0)
    ec1 = _edge_prep(W_m1[H:], bm1)

    sc_round = _make_sc_round(n_nodes, n_edges)

    parts0 = sc_round(p0, ec0, src, dst).reshape(NC, n_nodes, 2 * H)

    ns1, p1 = pl.pallas_call(
        _update_body,
        out_shape=(jax.ShapeDtypeStruct((n_nodes, H), f32),
                   jax.ShapeDtypeStruct((n_nodes, 2 * H), f32)),
    )(ns0, parts0, W_u0[:H], W_u0[H:], bu0, wm1_top)

    parts1 = sc_round(p1, ec1, src, dst).reshape(NC, n_nodes, 2 * H)

    ns2 = pl.pallas_call(
        _final_body,
        out_shape=jax.ShapeDtypeStruct((n_nodes, H), f32),
    )(ns1, parts1, W_u1[:H], W_u1[H:], bu1)

    return ns2


# speedup vs baseline: 1.2671x; 1.2102x over previous
import functools


import jax
import jax.numpy as jnp
from jax import lax
from jax.experimental import pallas as pl
from jax.experimental.pallas import tpu as pltpu
from jax.experimental.pallas import tpu_sc as plsc

H = 64
L = 16
NC = 2
NS = 16
NW = NC * NS
SB = 80


def _node_prep_body(nf, w, b, wtop, ns_o, p_o):
    ns = jnp.maximum(jnp.dot(nf[...], w[...], preferred_element_type=jnp.float32)
                     + b[...], 0.0)
    ns_o[...] = ns
    p = jnp.dot(ns, wtop[...], preferred_element_type=jnp.float32)
    p_o[...] = jnp.concatenate([p, jnp.zeros_like(p)], axis=1)


def _edge_prep_body(ef_lo, ef_hi, we, be, wm, bm, ec_o):
    es_lo = jnp.maximum(
        jnp.dot(ef_lo[...], we[...], preferred_element_type=jnp.float32)
        + be[...], 0.0)
    es_hi = jnp.maximum(
        jnp.dot(ef_hi[...], we[...], preferred_element_type=jnp.float32)
        + be[...], 0.0)
    lo = jnp.dot(es_lo, wm[...], preferred_element_type=jnp.float32) + bm[...]
    hi = jnp.dot(es_hi, wm[...], preferred_element_type=jnp.float32) + bm[...]
    ec_o[...] = jnp.concatenate([lo, hi], axis=1)


def _update_body(ns, parts, wu_t, wu_b, bu, wtop, ns_o, p_o):
    agg = parts[0, :, :H] + parts[1, :, :H]
    x = (jnp.dot(ns[...], wu_t[...], preferred_element_type=jnp.float32)
         + jnp.dot(agg, wu_b[...], preferred_element_type=jnp.float32) + bu[...])
    ns2 = jnp.maximum(x, 0.0)
    ns_o[...] = ns2
    p = jnp.dot(ns2, wtop[...], preferred_element_type=jnp.float32)
    p_o[...] = jnp.concatenate([p, jnp.zeros_like(p)], axis=1)


def _final_body(ns, parts, wu_t, wu_b, bu, ns_o):
    agg = parts[0, :, :H] + parts[1, :, :H]
    x = (jnp.dot(ns[...], wu_t[...], preferred_element_type=jnp.float32)
         + jnp.dot(agg, wu_b[...], preferred_element_type=jnp.float32) + bu[...])
    ns_o[...] = jnp.maximum(x, 0.0)


def _make_sc_round(n_nodes, n_edges):
    edges_per_tile = n_edges // NW
    rows_per_tile = n_nodes // NS
    mesh = plsc.VectorSubcoreMesh(core_axis_name="c", subcore_axis_name="s",
                                  num_cores=NC, num_subcores=NS)

    n_iter = edges_per_tile // SB
    PR = SB // 2

    @functools.partial(
        pl.kernel, mesh=mesh,
        compiler_params=pltpu.CompilerParams(needs_layout_passes=False),
        out_type=jax.ShapeDtypeStruct((NW, n_nodes // NS, 2 * H), jnp.float32),
        scratch_types=[
            pltpu.VMEM_SHARED((n_nodes, 2 * H), jnp.float32),
            [pltpu.VMEM((PR, 2 * H), jnp.float32) for _ in range(3)],
            [pltpu.VMEM((SB, 2 * H), jnp.float32) for _ in range(3)],
            [pltpu.VMEM((SB,), jnp.int32) for _ in range(3)],
            [pltpu.VMEM((SB,), jnp.int32) for _ in range(6)],
            [pltpu.SemaphoreType.DMA for _ in range(3)],
            [pltpu.SemaphoreType.DMA for _ in range(3)],
            [pltpu.SemaphoreType.DMA for _ in range(3)],
            [pltpu.SemaphoreType.DMA for _ in range(3)],
        ],
    )
    def sc_round(p_hbm, ec_hbm, src_hbm, dst_hbm, out_hbm,
                 agg_sh, ec_v, rows_v, sidx, didx,
                 sem_ec, sem_g, sem_i, sem_s):
        cid = lax.axis_index("c")
        sid = lax.axis_index("s")
        wid = sid * NC + cid

        zero = jnp.zeros((L,), jnp.float32)

        def _zstep(i, carry):
            for j in range(2 * H // L):
                rows_v[0][i, pl.ds(j * L, L)] = zero
            return carry

        lax.fori_loop(0, SB, _zstep, 0)
        node_base = sid * rows_per_tile
        done = 0
        while done < rows_per_tile:
            n = min(SB, rows_per_tile - done)
            pltpu.sync_copy(rows_v[0].at[pl.ds(0, n)],
                            agg_sh.at[pl.ds(node_base + done, n)])
            done += n
        plsc.subcore_barrier()

        pair0 = wid * (edges_per_tile // 2)
        half = n_edges // 2

        def issue_stage(k, b, d):
            lo = pair0 + k * PR
            pltpu.async_copy(src_hbm.at[pl.ds(lo, PR)],
                             sidx[b].at[pl.ds(0, PR)], sem_i[b])
            pltpu.async_copy(src_hbm.at[pl.ds(half + lo, PR)],
                             sidx[b].at[pl.ds(PR, PR)], sem_i[b])
            pltpu.async_copy(dst_hbm.at[pl.ds(lo, PR)],
                             didx[d].at[pl.ds(0, PR)], sem_i[b])
            pltpu.async_copy(dst_hbm.at[pl.ds(half + lo, PR)],
                             didx[d].at[pl.ds(PR, PR)], sem_i[b])
            pltpu.async_copy(ec_hbm.at[pl.ds(lo, PR)], ec_v[b], sem_ec[b])

        def wait_stage_idx(b):
            pltpu.make_async_copy(src_hbm.at[pl.ds(0, SB)], sidx[b],
                                  sem_i[b]).wait()
            pltpu.make_async_copy(dst_hbm.at[pl.ds(0, SB)], didx[0],
                                  sem_i[b]).wait()

        def wait_ec(b):
            pltpu.make_async_copy(ec_hbm.at[pl.ds(0, PR)], ec_v[b],
                                  sem_ec[b]).wait()

        def issue_gather(b):
            pltpu.async_copy(p_hbm.at[sidx[b]], rows_v[b], sem_g[b])

        def wait_gather(b):
            pltpu.make_async_copy(p_hbm.at[sidx[b]], rows_v[b],
                                  sem_g[b]).wait()

        def issue_scatter(b, d):
            pltpu.async_copy(rows_v[b], agg_sh.at[didx[d]], sem_s[b], add=True)

        def wait_scatter(b):
            pltpu.make_async_copy(rows_v[b], agg_sh.at[didx[0]],
                                  sem_s[b]).wait()

        def compute(b):
            ecb = ec_v[b]
            rb = rows_v[b]

            @plsc.parallel_loop(0, PR, 1, unroll=8)
            def _row(i2):
                for jj in range(2 * H // L):
                    se = pl.ds(jj * L, L)
                    sr = pl.ds((jj % 4) * L, L)
                    r = i2 + PR * (jj // 4)
                    rb[r, sr] = jnp.maximum(ecb[i2, se] + rb[r, sr], 0.0)

        def iteration(k, p, do_g=True, do_a=True, first_g=False):
            r = p % 3
            r2 = (p + 2) % 3
            wait_ec(r)
            wait_gather(r)
            compute(r)
            issue_scatter(r, p)
            if do_g:
                wait_stage_idx(r2)
                if not first_g:
                    wait_scatter(r2)
                issue_gather(r2)
            if do_a:
                issue_stage(k + 3, r, (p + 3) % 6)

        for kk in range(3):
            issue_stage(kk, kk, kk)
        wait_stage_idx(0)
        issue_gather(0)
        wait_stage_idx(1)
        issue_gather(1)
        iteration(0, 0, first_g=True)
        iteration(1, 1)

        n_steady = n_iter - 5
        def _six(i, carry):
            k = 6 * i + 2
            for j in range(6):
                iteration(k + j, (2 + j) % 6)
            return carry

        lax.fori_loop(0, n_steady // 6, _six, 0)
        iteration(n_iter - 3, (n_iter - 3) % 6, do_a=False)
        iteration(n_iter - 2, (n_iter - 2) % 6, do_g=False, do_a=False)
        iteration(n_iter - 1, (n_iter - 1) % 6, do_g=False, do_a=False)
        for kk in (n_iter - 3, n_iter - 2, n_iter - 1):
            wait_scatter(kk % 3)
        plsc.subcore_barrier()

        pltpu.sync_copy(agg_sh.at[pl.ds(node_base, rows_per_tile)],
                        out_hbm.at[cid * NS + sid])

    return sc_round


def kernel(node_features, edge_index, edge_features,
           W_node_in, b_node_in, W_edge_in, b_edge_in,
           W_m0, b_m0, W_m1, b_m1, W_u0, b_u0, W_u1, b_u1):
    n_nodes, _ = node_features.shape
    n_edges = edge_index.shape[1]
    f32 = jnp.float32

    src = edge_index[0]
    dst = edge_index[1]
    b_node = b_node_in.reshape(1, H)
    b_edge = b_edge_in.reshape(1, H)
    bm0 = b_m0.reshape(1, H)
    bm1 = b_m1.reshape(1, H)
    bu0 = b_u0.reshape(1, H)
    bu1 = b_u1.reshape(1, H)
    d_edge = edge_features.shape[1]

    wm0_top = W_m0[:H]
    wm1_top = W_m1[:H]

    ns0, p0 = pl.pallas_call(
        _node_prep_body,
        out_shape=(jax.ShapeDtypeStruct((n_nodes, H), f32),
                   jax.ShapeDtypeStruct((n_nodes, 2 * H), f32)),
    )(node_features, W_node_in, b_node, wm0_top)

    eblk = 8000
    n_pairs = n_edges // 2
    grid = n_pairs // eblk
    nblk = grid

    def _edge_prep(wm_bot, bm_r):
        return pl.pallas_call(
            _edge_prep_body,
            grid=(grid,),
            in_specs=[
                pl.BlockSpec((eblk, d_edge), lambda i: (i, 0)),
                pl.BlockSpec((eblk, d_edge), lambda i: (i + nblk, 0)),
                pl.BlockSpec((d_edge, H), lambda i: (0, 0)),
                pl.BlockSpec((1, H), lambda i: (0, 0)),
                pl.BlockSpec((H, H), lambda i: (0, 0)),
                pl.BlockSpec((1, H), lambda i: (0, 0)),
            ],
            out_specs=pl.BlockSpec((eblk, 2 * H), lambda i: (i, 0)),
            out_shape=jax.ShapeDtypeStruct((n_pairs, 2 * H), f32),
        )(edge_features, edge_features, W_edge_in, b_edge, wm_bot, bm_r)

    ec0 = _edge_prep(W_m0[H:], bm0)
    ec1 = _edge_prep(W_m1[H:], bm1)

    sc_round = _make_sc_round(n_nodes, n_edges)

    parts0 = sc_round(p0, ec0, src, dst).reshape(NC, n_nodes, 2 * H)

    ns1, p1 = pl.pallas_call(
        _update_body,
        out_shape=(jax.ShapeDtypeStruct((n_nodes, H), f32),
                   jax.ShapeDtypeStruct((n_nodes, 2 * H), f32)),
    )(ns0, parts0, W_u0[:H], W_u0[H:], bu0, wm1_top)

    parts1 = sc_round(p1, ec1, src, dst).reshape(NC, n_nodes, 2 * H)

    ns2 = pl.pallas_call(
        _final_body,
        out_shape=jax.ShapeDtypeStruct((n_nodes, H), f32),
    )(ns1, parts1, W_u1[:H], W_u1[H:], bu1)

    return ns2


# speedup vs baseline: 1.2680x; 1.0007x over previous
import functools


import jax
import jax.numpy as jnp
from jax import lax
from jax.experimental import pallas as pl
from jax.experimental.pallas import tpu as pltpu
from jax.experimental.pallas import tpu_sc as plsc

H = 64
L = 16
NC = 2
NS = 16
NW = NC * NS
SB = 80


def _prep_body(ef_lo, ef_hi, we, be, wm, bm, nf, w, b, wtop, ec_o, ns_o, p_o):
    es_lo = jnp.maximum(
        jnp.dot(ef_lo[...], we[...], preferred_element_type=jnp.float32)
        + be[...], 0.0)
    es_hi = jnp.maximum(
        jnp.dot(ef_hi[...], we[...], preferred_element_type=jnp.float32)
        + be[...], 0.0)
    lo = jnp.dot(es_lo, wm[...], preferred_element_type=jnp.float32) + bm[...]
    hi = jnp.dot(es_hi, wm[...], preferred_element_type=jnp.float32) + bm[...]
    ec_o[...] = jnp.concatenate([lo, hi], axis=1)

    @pl.when(pl.program_id(0) == 0)
    def _node_prep():
        ns = jnp.maximum(
            jnp.dot(nf[...], w[...], preferred_element_type=jnp.float32)
            + b[...], 0.0)
        ns_o[...] = ns
        p = jnp.dot(ns, wtop[...], preferred_element_type=jnp.float32)
        p_o[...] = jnp.concatenate([p, jnp.zeros_like(p)], axis=1)


def _edge_prep_body(ef_lo, ef_hi, we, be, wm, bm, ec_o):
    es_lo = jnp.maximum(
        jnp.dot(ef_lo[...], we[...], preferred_element_type=jnp.float32)
        + be[...], 0.0)
    es_hi = jnp.maximum(
        jnp.dot(ef_hi[...], we[...], preferred_element_type=jnp.float32)
        + be[...], 0.0)
    lo = jnp.dot(es_lo, wm[...], preferred_element_type=jnp.float32) + bm[...]
    hi = jnp.dot(es_hi, wm[...], preferred_element_type=jnp.float32) + bm[...]
    ec_o[...] = jnp.concatenate([lo, hi], axis=1)


def _update_body(ns, parts, wu_t, wu_b, bu, wtop, ns_o, p_o):
    agg = parts[0, :, :H] + parts[1, :, :H]
    x = (jnp.dot(ns[...], wu_t[...], preferred_element_type=jnp.float32)
         + jnp.dot(agg, wu_b[...], preferred_element_type=jnp.float32) + bu[...])
    ns2 = jnp.maximum(x, 0.0)
    ns_o[...] = ns2
    p = jnp.dot(ns2, wtop[...], preferred_element_type=jnp.float32)
    p_o[...] = jnp.concatenate([p, jnp.zeros_like(p)], axis=1)


def _final_body(ns, parts, wu_t, wu_b, bu, ns_o):
    agg = parts[0, :, :H] + parts[1, :, :H]
    x = (jnp.dot(ns[...], wu_t[...], preferred_element_type=jnp.float32)
         + jnp.dot(agg, wu_b[...], preferred_element_type=jnp.float32) + bu[...])
    ns_o[...] = jnp.maximum(x, 0.0)


def _make_sc_round(n_nodes, n_edges):
    edges_per_tile = n_edges // NW
    rows_per_tile = n_nodes // NS
    mesh = plsc.VectorSubcoreMesh(core_axis_name="c", subcore_axis_name="s",
                                  num_cores=NC, num_subcores=NS)

    n_iter = edges_per_tile // SB
    PR = SB // 2

    @functools.partial(
        pl.kernel, mesh=mesh,
        compiler_params=pltpu.CompilerParams(needs_layout_passes=False),
        out_type=jax.ShapeDtypeStruct((NW, n_nodes // NS, 2 * H), jnp.float32),
        scratch_types=[
            pltpu.VMEM_SHARED((n_nodes, 2 * H), jnp.float32),
            [pltpu.VMEM((PR, 2 * H), jnp.float32) for _ in range(3)],
            [pltpu.VMEM((SB, 2 * H), jnp.float32) for _ in range(3)],
            [pltpu.VMEM((SB,), jnp.int32) for _ in range(3)],
            [pltpu.VMEM((SB,), jnp.int32) for _ in range(6)],
            [pltpu.SemaphoreType.DMA for _ in range(3)],
            [pltpu.SemaphoreType.DMA for _ in range(3)],
            [pltpu.SemaphoreType.DMA for _ in range(3)],
            [pltpu.SemaphoreType.DMA for _ in range(3)],
        ],
    )
    def sc_round(p_hbm, ec_hbm, src_hbm, dst_hbm, out_hbm,
                 agg_sh, ec_v, rows_v, sidx, didx,
                 sem_ec, sem_g, sem_i, sem_s):
        cid = lax.axis_index("c")
        sid = lax.axis_index("s")
        wid = sid * NC + cid

        zero = jnp.zeros((L,), jnp.float32)

        def _zstep(i, carry):
            for j in range(2 * H // L):
                rows_v[0][i, pl.ds(j * L, L)] = zero
            return carry

        lax.fori_loop(0, SB, _zstep, 0)
        node_base = sid * rows_per_tile
        done = 0
        while done < rows_per_tile:
            n = min(SB, rows_per_tile - done)
            pltpu.sync_copy(rows_v[0].at[pl.ds(0, n)],
                            agg_sh.at[pl.ds(node_base + done, n)])
            done += n
        plsc.subcore_barrier()

        pair0 = wid * (edges_per_tile // 2)
        half = n_edges // 2

        def issue_stage(k, b, d):
            lo = pair0 + k * PR
            pltpu.async_copy(src_hbm.at[pl.ds(lo, PR)],
                             sidx[b].at[pl.ds(0, PR)], sem_i[b])
            pltpu.async_copy(src_hbm.at[pl.ds(half + lo, PR)],
                             sidx[b].at[pl.ds(PR, PR)], sem_i[b])
            pltpu.async_copy(dst_hbm.at[pl.ds(lo, PR)],
                             didx[d].at[pl.ds(0, PR)], sem_i[b])
            pltpu.async_copy(dst_hbm.at[pl.ds(half + lo, PR)],
                             didx[d].at[pl.ds(PR, PR)], sem_i[b])
            pltpu.async_copy(ec_hbm.at[pl.ds(lo, PR)], ec_v[b], sem_ec[b])

        def wait_stage_idx(b):
            pltpu.make_async_copy(src_hbm.at[pl.ds(0, SB)], sidx[b],
                                  sem_i[b]).wait()
            pltpu.make_async_copy(dst_hbm.at[pl.ds(0, SB)], didx[0],
                                  sem_i[b]).wait()

        def wait_ec(b):
            pltpu.make_async_copy(ec_hbm.at[pl.ds(0, PR)], ec_v[b],
                                  sem_ec[b]).wait()

        def issue_gather(b):
            pltpu.async_copy(p_hbm.at[sidx[b]], rows_v[b], sem_g[b])

        def wait_gather(b):
            pltpu.make_async_copy(p_hbm.at[sidx[b]], rows_v[b],
                                  sem_g[b]).wait()

        def issue_scatter(b, d):
            pltpu.async_copy(rows_v[b], agg_sh.at[didx[d]], sem_s[b], add=True)

        def wait_scatter(b):
            pltpu.make_async_copy(rows_v[b], agg_sh.at[didx[0]],
                                  sem_s[b]).wait()

        def compute(b):
            ecb = ec_v[b]
            rb = rows_v[b]

            @plsc.parallel_loop(0, PR, 1, unroll=8)
            def _row(i2):
                for jj in range(2 * H // L):
                    se = pl.ds(jj * L, L)
                    sr = pl.ds((jj % 4) * L, L)
                    r = i2 + PR * (jj // 4)
                    rb[r, sr] = jnp.maximum(ecb[i2, se] + rb[r, sr], 0.0)

        def iteration(k, p, do_g=True, do_a=True, first_g=False):
            r = p % 3
            r2 = (p + 2) % 3
            wait_ec(r)
            wait_gather(r)
            compute(r)
            issue_scatter(r, p)
            if do_g:
                wait_stage_idx(r2)
                if not first_g:
                    wait_scatter(r2)
                issue_gather(r2)
            if do_a:
                issue_stage(k + 3, r, (p + 3) % 6)

        for kk in range(3):
            issue_stage(kk, kk, kk)
        wait_stage_idx(0)
        issue_gather(0)
        wait_stage_idx(1)
        issue_gather(1)
        iteration(0, 0, first_g=True)
        iteration(1, 1)

        n_steady = n_iter - 5
        def _six(i, carry):
            k = 6 * i + 2
            for j in range(6):
                iteration(k + j, (2 + j) % 6)
            return carry

        lax.fori_loop(0, n_steady // 6, _six, 0)
        iteration(n_iter - 3, (n_iter - 3) % 6, do_a=False)
        iteration(n_iter - 2, (n_iter - 2) % 6, do_g=False, do_a=False)
        iteration(n_iter - 1, (n_iter - 1) % 6, do_g=False, do_a=False)
        for kk in (n_iter - 3, n_iter - 2, n_iter - 1):
            wait_scatter(kk % 3)
        plsc.subcore_barrier()

        pltpu.sync_copy(agg_sh.at[pl.ds(node_base, rows_per_tile)],
                        out_hbm.at[cid * NS + sid])

    return sc_round


def kernel(node_features, edge_index, edge_features,
           W_node_in, b_node_in, W_edge_in, b_edge_in,
           W_m0, b_m0, W_m1, b_m1, W_u0, b_u0, W_u1, b_u1):
    n_nodes, _ = node_features.shape
    n_edges = edge_index.shape[1]
    f32 = jnp.float32

    src = edge_index[0]
    dst = edge_index[1]
    b_node = b_node_in.reshape(1, H)
    b_edge = b_edge_in.reshape(1, H)
    bm0 = b_m0.reshape(1, H)
    bm1 = b_m1.reshape(1, H)
    bu0 = b_u0.reshape(1, H)
    bu1 = b_u1.reshape(1, H)
    d_edge = edge_features.shape[1]

    wm0_top = W_m0[:H]
    wm1_top = W_m1[:H]

    eblk = 8000
    n_pairs = n_edges // 2
    grid = n_pairs // eblk
    nblk = grid

    ec0, ns0, p0 = pl.pallas_call(
        _prep_body,
        grid=(grid,),
        in_specs=[
            pl.BlockSpec((eblk, d_edge), lambda i: (i, 0)),
            pl.BlockSpec((eblk, d_edge), lambda i: (i + nblk, 0)),
            pl.BlockSpec((d_edge, H), lambda i: (0, 0)),
            pl.BlockSpec((1, H), lambda i: (0, 0)),
            pl.BlockSpec((H, H), lambda i: (0, 0)),
            pl.BlockSpec((1, H), lambda i: (0, 0)),
            pl.BlockSpec(node_features.shape, lambda i: (0, 0)),
            pl.BlockSpec(W_node_in.shape, lambda i: (0, 0)),
            pl.BlockSpec((1, H), lambda i: (0, 0)),
            pl.BlockSpec((H, H), lambda i: (0, 0)),
        ],
        out_specs=(pl.BlockSpec((eblk, 2 * H), lambda i: (i, 0)),
                   pl.BlockSpec((n_nodes, H), lambda i: (0, 0)),
                   pl.BlockSpec((n_nodes, 2 * H), lambda i: (0, 0))),
        out_shape=(jax.ShapeDtypeStruct((n_pairs, 2 * H), f32),
                   jax.ShapeDtypeStruct((n_nodes, H), f32),
                   jax.ShapeDtypeStruct((n_nodes, 2 * H), f32)),
    )(edge_features, edge_features, W_edge_in, b_edge, W_m0[H:], bm0,
      node_features, W_node_in, b_node, wm0_top)

    ec1 = pl.pallas_call(
        _edge_prep_body,
        grid=(grid,),
        in_specs=[
            pl.BlockSpec((eblk, d_edge), lambda i: (i, 0)),
            pl.BlockSpec((eblk, d_edge), lambda i: (i + nblk, 0)),
            pl.BlockSpec((d_edge, H), lambda i: (0, 0)),
            pl.BlockSpec((1, H), lambda i: (0, 0)),
            pl.BlockSpec((H, H), lambda i: (0, 0)),
            pl.BlockSpec((1, H), lambda i: (0, 0)),
        ],
        out_specs=pl.BlockSpec((eblk, 2 * H), lambda i: (i, 0)),
        out_shape=jax.ShapeDtypeStruct((n_pairs, 2 * H), f32),
    )(edge_features, edge_features, W_edge_in, b_edge, W_m1[H:], bm1)

    sc_round = _make_sc_round(n_nodes, n_edges)

    parts0 = sc_round(p0, ec0, src, dst).reshape(NC, n_nodes, 2 * H)

    ns1, p1 = pl.pallas_call(
        _update_body,
        out_shape=(jax.ShapeDtypeStruct((n_nodes, H), f32),
                   jax.ShapeDtypeStruct((n_nodes, 2 * H), f32)),
    )(ns0, parts0, W_u0[:H], W_u0[H:], bu0, wm1_top)

    parts1 = sc_round(p1, ec1, src, dst).reshape(NC, n_nodes, 2 * H)

    ns2 = pl.pallas_call(
        _final_body,
        out_shape=jax.ShapeDtypeStruct((n_nodes, H), f32),
    )(ns1, parts1, W_u1[:H], W_u1[H:], bu1)

    return ns2
